# Initial kernel scaffold; baseline (speedup 1.0000x reference)
import jax
import jax.numpy as jnp
from jax.experimental import pallas as pl


def kernel(atoms, bonds, edges, W, b):
    raise NotImplementedError("write your pallas kernel here")


# speedup vs baseline: 43.1655x; 43.1655x over previous
import jax
import jax.numpy as jnp
from jax.experimental import pallas as pl

_B, _A, _F = 1024, 64, 128
_D, _FB, _CW = 6, 4, 32
_G = 8
_GA = _G * _A


def _tc_body(edges_ref, atoms_ref, bonds_ref, wa_ref, wb_ref, bias_ref, out_ref):
    atoms = atoms_ref[...]
    edges = edges_ref[...]
    deg = jnp.sum((edges != -1).astype(jnp.float32), axis=1, keepdims=True)

    iota_row = jax.lax.broadcasted_iota(jnp.int32, (_A, _A), 1)
    iota_col = jax.lax.broadcasted_iota(jnp.int32, (_A, _A), 0)
    eye = (iota_col == iota_row).astype(jnp.float32)

    sa_parts = []
    for g in range(_G):
        e_g = edges[g * _A:(g + 1) * _A, :]
        adj = eye
        for d in range(_D):
            adj = adj + (e_g[:, d:d + 1] == iota_row).astype(jnp.float32)
        sa_parts.append(
            jnp.dot(adj, atoms[g * _A:(g + 1) * _A, :],
                    preferred_element_type=jnp.float32))
    sa = jnp.concatenate(sa_parts, axis=0)

    bonds = bonds_ref[...]
    sb = bonds[:, 0:_FB]
    for d in range(1, _D):
        sb = sb + bonds[:, d * _FB:(d + 1) * _FB]

    y = jnp.dot(sa, wa_ref[...], preferred_element_type=jnp.float32)
    y = y + jnp.dot(sb, wb_ref[...], preferred_element_type=jnp.float32)
    y = y + bias_ref[...]

    out = jnp.zeros((_GA, _CW), jnp.float32)
    for d in range(_D):
        mask = (deg == d).astype(jnp.float32)
        out = out + y[:, d * _CW:(d + 1) * _CW] * mask
    out_ref[...] = out


def kernel(atoms, bonds, edges, W, b):
    atoms2 = atoms.reshape(_B * _A, _F)
    bonds2 = bonds.reshape(_B * _A, _D * _FB)
    edges2 = edges.reshape(_B * _A, _D)
    wa = W[:, :_F, :].transpose(1, 0, 2).reshape(_F, _D * _CW)
    wb = W[:, _F:, :].transpose(1, 0, 2).reshape(_FB, _D * _CW)
    bias = b.reshape(1, _D * _CW)

    out = pl.pallas_call(
        _tc_body,
        grid=(_B // _G,),
        in_specs=[
            pl.BlockSpec((_GA, _D), lambda i: (i, 0)),
            pl.BlockSpec((_GA, _F), lambda i: (i, 0)),
            pl.BlockSpec((_GA, _D * _FB), lambda i: (i, 0)),
            pl.BlockSpec((_F, _D * _CW), lambda i: (0, 0)),
            pl.BlockSpec((_FB, _D * _CW), lambda i: (0, 0)),
            pl.BlockSpec((1, _D * _CW), lambda i: (0, 0)),
        ],
        out_specs=pl.BlockSpec((_GA, _CW), lambda i: (i, 0)),
        out_shape=jax.ShapeDtypeStruct((_B * _A, _CW), jnp.float32),
    )(edges2, atoms2, bonds2, wa, wb, bias)
    return out.reshape(_B, _A, _CW)


# speedup vs baseline: 48.3210x; 1.1194x over previous
import jax
import jax.numpy as jnp
import numpy as np
from jax.experimental import pallas as pl

_B, _A, _F = 1024, 64, 128
_D, _FB, _CW = 6, 4, 32
_G = 8
_GA = _G * _A
_PW = 2 * _A


def _tc_body(edges_ref, atoms_ref, bonds_ref, wa_ref, wb_ref, bias_ref,
             sel_ref, red_ref, out_ref):
    edges = edges_ref[...]
    atoms = atoms_ref[...]
    bonds = bonds_ref[...]
    deg = jnp.sum((edges != -1).astype(jnp.float32), axis=1, keepdims=True)

    iota_row = jax.lax.broadcasted_iota(jnp.int32, (_PW, _PW), 1)
    iota_col = jax.lax.broadcasted_iota(jnp.int32, (_PW, _PW), 0)
    eye = (iota_col == iota_row).astype(jnp.float32)

    for p in range(_GA // _PW):
        sl = slice(p * _PW, (p + 1) * _PW)
        e_p = edges[sl, :]
        adj = eye
        for d in range(_D):
            adj = adj + (e_p[:, d:d + 1] == iota_row).astype(jnp.float32)
        sa = jnp.dot(adj, atoms[sl, :], preferred_element_type=jnp.float32)
        y = jnp.dot(sa, wa_ref[...], preferred_element_type=jnp.float32)
        y = y + jnp.dot(bonds[sl, :], wb_ref[...],
                        preferred_element_type=jnp.float32)
        y = (y + bias_ref[...]) * (deg[sl, :] == sel_ref[...]).astype(jnp.float32)
        out_ref[sl, :] = jnp.dot(y, red_ref[...],
                                 preferred_element_type=jnp.float32)


def kernel(atoms, bonds, edges, W, b):
    atoms2 = atoms.reshape(_B * _A, _F)
    bonds2 = bonds.reshape(_B * _A, _D * _FB)
    odd = (jnp.arange(_B, dtype=jnp.int32) & 1).reshape(_B, 1, 1)
    edges2 = jnp.where(edges >= 0, edges + _A * odd, -1).reshape(_B * _A, _D)
    wa = W[:, :_F, :].transpose(1, 0, 2).reshape(_F, _D * _CW)
    wb = jnp.tile(W[:, _F:, :].transpose(1, 0, 2).reshape(_FB, _D * _CW),
                  (_D, 1))
    bias = b.reshape(1, _D * _CW)
    sel = jnp.asarray(np.repeat(np.arange(_D, dtype=np.float32), _CW)
                      ).reshape(1, _D * _CW)
    red = jnp.asarray(
        (np.arange(_D * _CW)[:, None] % _CW == np.arange(_CW)[None, :])
        .astype(np.float32))

    out = pl.pallas_call(
        _tc_body,
        grid=(_B // _G,),
        in_specs=[
            pl.BlockSpec((_GA, _D), lambda i: (i, 0)),
            pl.BlockSpec((_GA, _F), lambda i: (i, 0)),
            pl.BlockSpec((_GA, _D * _FB), lambda i: (i, 0)),
            pl.BlockSpec((_F, _D * _CW), lambda i: (0, 0)),
            pl.BlockSpec((_D * _FB, _D * _CW), lambda i: (0, 0)),
            pl.BlockSpec((1, _D * _CW), lambda i: (0, 0)),
            pl.BlockSpec((1, _D * _CW), lambda i: (0, 0)),
            pl.BlockSpec((_D * _CW, _CW), lambda i: (0, 0)),
        ],
        out_specs=pl.BlockSpec((_GA, _CW), lambda i: (i, 0)),
        out_shape=jax.ShapeDtypeStruct((_B * _A, _CW), jnp.float32),
    )(edges2, atoms2, bonds2, wa, wb, bias, sel, red)
    return out.reshape(_B, _A, _CW)
